# baseline (speedup 1.0000x reference)
import functools
import jax
import jax.numpy as jnp
from jax import lax
from jax.experimental import pallas as pl
from jax.experimental.pallas import tpu as pltpu
from jax.experimental.pallas import tpu_sc as plsc

_N = 100000
_HID = 128
_NSEG = 64
_NW = 32
_RW = 3128
_CR = 128
_TRIPS = 25
_NEG = -3.0e38


def _rmw_max(tab, idx, val):
    vec = tab[pl.ds(idx, 16)]
    lane0 = lax.broadcasted_iota(jnp.int32, (16,), 0) == 0
    tab[pl.ds(idx, 16)] = jnp.where(lane0, jnp.maximum(vec, jnp.full((16,), val)), vec)


def _rmw_add(tab, idx, val):
    vec = tab[pl.ds(idx, 16)]
    lane0 = lax.broadcasted_iota(jnp.int32, (16,), 0) == 0
    tab[pl.ds(idx, 16)] = jnp.where(lane0, vec + jnp.full((16,), val), vec)


def _sc_body(h_hbm, seg_hbm, q_hbm, w_hbm, m_hbm, s_hbm, acc_hbm,
             hbuf, segbuf, gbuf, qv, wv, qdotv, m_tab, s_tab, acc_tab):
    wid = lax.axis_index("s") * 2 + lax.axis_index("c")
    base = wid * _RW
    end = jnp.minimum(base + _RW, _N)
    iota = lax.broadcasted_iota(jnp.int32, (16,), 0)
    zero16 = jnp.zeros((16,), jnp.float32)
    neg16 = jnp.full((16,), _NEG, jnp.float32)

    pltpu.sync_copy(w_hbm, wv)
    pltpu.sync_copy(q_hbm, qv)

    def qdot_grp(t, _):
        ridx = (t * 16 + iota) * _HID
        acc = zero16
        for k16 in range(_HID // 16):
            wvec = wv[pl.ds(_HID + k16 * 16, 16)]
            for dk in range(16):
                k = k16 * 16 + dk
                col = plsc.load_gather(qv, [ridx + k])
                acc = acc + col * jnp.full((16,), wvec[dk])
        qdotv[pl.ds(t * 16, 16)] = acc
        return 0

    lax.fori_loop(0, _NSEG // 16, qdot_grp, 0)

    for t in range(8):
        m_tab[pl.ds(t * 16, 16)] = neg16
        s_tab[pl.ds(t * 16, 16)] = zero16

    def acc_init(r, _):
        acc_tab[pl.ds(r * 16, 16)] = zero16
        return 0

    lax.fori_loop(0, _NSEG * _HID // 16, acc_init, 0)

    def p1_chunk(c, _):
        lo = base + c * _CR
        st = jnp.minimum(lo, _N - _CR)
        pltpu.sync_copy(h_hbm.at[pl.ds(st * _HID, _CR * _HID)], hbuf)
        pltpu.sync_copy(seg_hbm.at[pl.ds(st, _CR)], segbuf)

        def p1_grp(t, _):
            rows = t * 16 + iota
            ridx = rows * _HID
            hd = zero16
            for k16 in range(_HID // 16):
                wvec = wv[pl.ds(k16 * 16, 16)]
                for dk in range(16):
                    k = k16 * 16 + dk
                    col = plsc.load_gather(hbuf, [ridx + k])
                    hd = hd + col * jnp.full((16,), wvec[dk])
            seg16 = segbuf[pl.ds(t * 16, 16)]
            qd16 = plsc.load_gather(qdotv, [seg16])
            g16 = hd + qd16
            g16 = jnp.where(g16 >= 0, g16, 0.01 * g16)
            gl = st + rows
            valid = (gl >= lo) & (gl < end)
            g16 = jnp.where(valid, g16, _NEG)
            gbuf[pl.ds(c * _CR + t * 16, 16)] = g16

            s0 = seg16[0]
            s15 = seg16[15]

            def fast(_):
                _rmw_max(m_tab, s0, jnp.max(g16))
                return 0

            def slow(_):
                for j in range(16):
                    _rmw_max(m_tab, seg16[j], g16[j])
                return 0

            lax.cond(s0 == s15, fast, slow, 0)
            return 0

        lax.fori_loop(0, _CR // 16, p1_grp, 0)
        return 0

    lax.fori_loop(0, _TRIPS, p1_chunk, 0)

    def p2_chunk(c, _):
        lo = base + c * _CR
        st = jnp.minimum(lo, _N - _CR)
        pltpu.sync_copy(h_hbm.at[pl.ds(st * _HID, _CR * _HID)], hbuf)
        pltpu.sync_copy(seg_hbm.at[pl.ds(st, _CR)], segbuf)

        def p2_grp(t, _):
            rows = t * 16 + iota
            seg16 = segbuf[pl.ds(t * 16, 16)]
            g16 = gbuf[pl.ds(c * _CR + t * 16, 16)]
            m16 = plsc.load_gather(m_tab, [seg16])
            e16 = jnp.exp(g16 - m16)
            gl = st + rows
            valid = (gl >= lo) & (gl < end)
            e16 = jnp.where(valid, e16, 0.0)

            s0 = seg16[0]
            s15 = seg16[15]

            def fast(_):
                _rmw_add(s_tab, s0, jnp.sum(e16))
                arow = s0 * _HID
                acc = [acc_tab[pl.ds(arow + k * 16, 16)]
                       for k in range(_HID // 16)]
                for j in range(16):
                    ev = jnp.full((16,), e16[j])
                    hrow = (t * 16 + j) * _HID
                    for k in range(_HID // 16):
                        acc[k] = acc[k] + hbuf[pl.ds(hrow + k * 16, 16)] * ev
                for k in range(_HID // 16):
                    acc_tab[pl.ds(arow + k * 16, 16)] = acc[k]
                return 0

            def slow(_):
                for j in range(16):
                    sj = seg16[j]
                    ev = jnp.full((16,), e16[j])
                    _rmw_add(s_tab, sj, e16[j])
                    arow = sj * _HID
                    hrow = (t * 16 + j) * _HID
                    for k in range(_HID // 16):
                        acc_tab[pl.ds(arow + k * 16, 16)] = (
                            acc_tab[pl.ds(arow + k * 16, 16)]
                            + hbuf[pl.ds(hrow + k * 16, 16)] * ev)
                return 0

            lax.cond(s0 == s15, fast, slow, 0)
            return 0

        lax.fori_loop(0, _CR // 16, p2_grp, 0)
        return 0

    lax.fori_loop(0, _TRIPS, p2_chunk, 0)

    pltpu.sync_copy(m_tab, m_hbm.at[wid])
    pltpu.sync_copy(s_tab, s_hbm.at[wid])
    pltpu.sync_copy(acc_tab, acc_hbm.at[wid])


def _combine_body(m_ref, s_ref, acc_ref, out_ref):
    m = m_ref[:, :_NSEG]
    mg = jnp.max(m, axis=0)
    al = jnp.exp(m - mg[None, :])
    s = jnp.sum(s_ref[:, :_NSEG] * al, axis=0)
    acc = jnp.zeros((_NSEG, _HID), jnp.float32)
    for i in range(_NW):
        acc = acc + acc_ref[i] * al[i][:, None]
    out_ref[...] = jnp.where(s[:, None] > 0, acc / s[:, None], 0.0)


def kernel(h, segment_ids, attention_query, w):
    seg = segment_ids.astype(jnp.int32)
    wflat = w.reshape(2 * _HID)
    hflat = h.reshape(_N * _HID)

    mesh = plsc.VectorSubcoreMesh(core_axis_name="c", subcore_axis_name="s",
                                  num_cores=2, num_subcores=16)
    sck = functools.partial(
        pl.kernel,
        mesh=mesh,
        compiler_params=pltpu.CompilerParams(needs_layout_passes=False),
        out_type=[
            jax.ShapeDtypeStruct((_NW, _HID), jnp.float32),
            jax.ShapeDtypeStruct((_NW, _HID), jnp.float32),
            jax.ShapeDtypeStruct((_NW, _NSEG * _HID), jnp.float32),
        ],
        scratch_types=[
            pltpu.VMEM((_CR * _HID,), jnp.float32),
            pltpu.VMEM((_CR,), jnp.int32),
            pltpu.VMEM((_TRIPS * _CR,), jnp.float32),
            pltpu.VMEM((_NSEG * _HID,), jnp.float32),
            pltpu.VMEM((2 * _HID,), jnp.float32),
            pltpu.VMEM((_NSEG,), jnp.float32),
            pltpu.VMEM((_HID,), jnp.float32),
            pltpu.VMEM((_HID,), jnp.float32),
            pltpu.VMEM((_NSEG * _HID,), jnp.float32),
        ],
    )(_sc_body)
    m_p, s_p, acc_p = sck(hflat, seg, attention_query.reshape(_NSEG * _HID), wflat)

    return pl.pallas_call(
        _combine_body,
        out_shape=jax.ShapeDtypeStruct((_NSEG, _HID), jnp.float32),
    )(m_p, s_p, acc_p.reshape(_NW, _NSEG, _HID))


# speedup vs baseline: 1.3480x; 1.3480x over previous
import functools
import jax
import jax.numpy as jnp
from jax import lax
from jax.experimental import pallas as pl
from jax.experimental.pallas import tpu as pltpu
from jax.experimental.pallas import tpu_sc as plsc

_N = 100000
_HID = 128
_NSEG = 64
_NW = 32
_RW = 3128
_CR = 128
_TRIPS = 26
_NEG = -3.0e38


def _sc_body(h_hbm, seg_hbm, q_hbm, w_hbm, m_hbm, s_hbm, acc_hbm,
             hbuf0, hbuf1, segbuf0, segbuf1, qv, wv, qdotv,
             m_tab, s_tab, acc_tab, semh0, semh1, sems0, sems1):
    wid = lax.axis_index("s") * 2 + lax.axis_index("c")
    base = wid * _RW
    end = jnp.minimum(base + _RW, _N)
    iota = lax.broadcasted_iota(jnp.int32, (16,), 0)
    zero16 = jnp.zeros((16,), jnp.float32)
    neg16 = jnp.full((16,), _NEG, jnp.float32)
    lane0 = iota == 0

    pltpu.sync_copy(w_hbm, wv)
    pltpu.sync_copy(q_hbm, qv)

    def qdot_grp(t, _):
        ridx = (t * 16 + iota) * _HID
        a0 = zero16
        a1 = zero16
        for k16 in range(_HID // 16):
            wvec = wv[pl.ds(_HID + k16 * 16, 16)]
            for dk in range(16):
                k = k16 * 16 + dk
                col = plsc.load_gather(qv, [ridx + k])
                if k % 2 == 0:
                    a0 = a0 + col * jnp.full((16,), wvec[dk])
                else:
                    a1 = a1 + col * jnp.full((16,), wvec[dk])
        qdotv[pl.ds(t * 16, 16)] = a0 + a1
        return 0

    lax.fori_loop(0, _NSEG // 16, qdot_grp, 0)

    for t in range(8):
        m_tab[pl.ds(t * 16, 16)] = neg16
        s_tab[pl.ds(t * 16, 16)] = zero16

    def acc_init(r, _):
        acc_tab[pl.ds(r * 16, 16)] = zero16
        return 0

    lax.fori_loop(0, _NSEG * _HID // 16, acc_init, 0)

    def dma_start(c, hbuf, segbuf, semh, sems):
        lo = base + c * _CR
        st = jnp.minimum(lo, _N - _CR)
        pltpu.async_copy(h_hbm.at[pl.ds(st * _HID, _CR * _HID)], hbuf, semh)
        pltpu.async_copy(seg_hbm.at[pl.ds(st, _CR)], segbuf, sems)

    def dma_wait(c, hbuf, segbuf, semh, sems):
        lo = base + c * _CR
        st = jnp.minimum(lo, _N - _CR)
        pltpu.make_async_copy(h_hbm.at[pl.ds(st * _HID, _CR * _HID)], hbuf, semh).wait()
        pltpu.make_async_copy(seg_hbm.at[pl.ds(st, _CR)], segbuf, sems).wait()

    def process(c, hbuf, segbuf):
        lo = base + c * _CR
        st = jnp.minimum(lo, _N - _CR)

        def grp(t, _):
            rows = t * 16 + iota
            ridx = rows * _HID
            a0 = zero16
            a1 = zero16
            a2 = zero16
            a3 = zero16
            for k16 in range(_HID // 16):
                wvec = wv[pl.ds(k16 * 16, 16)]
                for dk in range(16):
                    k = k16 * 16 + dk
                    col = plsc.load_gather(hbuf, [ridx + k])
                    wk = jnp.full((16,), wvec[dk])
                    if k % 4 == 0:
                        a0 = a0 + col * wk
                    elif k % 4 == 1:
                        a1 = a1 + col * wk
                    elif k % 4 == 2:
                        a2 = a2 + col * wk
                    else:
                        a3 = a3 + col * wk
            seg16 = segbuf[pl.ds(t * 16, 16)]
            qd16 = plsc.load_gather(qdotv, [seg16])
            g16 = (a0 + a1) + (a2 + a3) + qd16
            g16 = jnp.where(g16 >= 0, g16, 0.01 * g16)
            gl = st + rows
            valid = (gl >= lo) & (gl < end)
            g16 = jnp.where(valid, g16, _NEG)
            vmask = jnp.where(valid, 1.0, 0.0)

            s0 = seg16[0]
            s15 = seg16[15]

            def fast(_):
                mvec = m_tab[pl.ds(s0, 16)]
                m_old = mvec[0]
                m_new = jnp.maximum(m_old, jnp.max(g16))
                av = jnp.exp(jnp.full((16,), m_old - m_new))
                e16 = jnp.exp(g16 - jnp.full((16,), m_new)) * vmask
                svec = s_tab[pl.ds(s0, 16)]
                s_tab[pl.ds(s0, 16)] = jnp.where(
                    lane0, svec * av + jnp.full((16,), jnp.sum(e16)), svec)
                m_tab[pl.ds(s0, 16)] = jnp.where(
                    lane0, jnp.full((16,), m_new), mvec)
                arow = s0 * _HID
                tmp = [zero16 for _ in range(_HID // 16)]
                for j in range(16):
                    ev = jnp.full((16,), e16[j])
                    hrow = (t * 16 + j) * _HID
                    for k in range(_HID // 16):
                        tmp[k] = tmp[k] + hbuf[pl.ds(hrow + k * 16, 16)] * ev
                for k in range(_HID // 16):
                    acc_tab[pl.ds(arow + k * 16, 16)] = (
                        acc_tab[pl.ds(arow + k * 16, 16)] * av + tmp[k])
                return 0

            def slow(_):
                for j in range(16):
                    sj = seg16[j]
                    gj = g16[j]
                    mvec = m_tab[pl.ds(sj, 16)]
                    m_old = mvec[0]
                    m_new = jnp.maximum(m_old, gj)
                    av = jnp.exp(jnp.full((16,), m_old - m_new))
                    ev = (jnp.exp(jnp.full((16,), gj - m_new))
                          * jnp.full((16,), vmask[j]))
                    svec = s_tab[pl.ds(sj, 16)]
                    s_tab[pl.ds(sj, 16)] = jnp.where(
                        lane0, svec * av + ev, svec)
                    m_tab[pl.ds(sj, 16)] = jnp.where(
                        lane0, jnp.full((16,), m_new), mvec)
                    arow = sj * _HID
                    hrow = (t * 16 + j) * _HID
                    for k in range(_HID // 16):
                        acc_tab[pl.ds(arow + k * 16, 16)] = (
                            acc_tab[pl.ds(arow + k * 16, 16)] * av
                            + hbuf[pl.ds(hrow + k * 16, 16)] * ev)
                return 0

            lax.cond(s0 == s15, fast, slow, 0)
            return 0

        lax.fori_loop(0, _CR // 16, grp, 0)

    dma_start(0, hbuf0, segbuf0, semh0, sems0)

    def ring(c2, _):
        c0 = 2 * c2
        dma_start(c0 + 1, hbuf1, segbuf1, semh1, sems1)
        dma_wait(c0, hbuf0, segbuf0, semh0, sems0)

        @pl.when(base + c0 * _CR < end)
        def _():
            process(c0, hbuf0, segbuf0)

        @pl.when(c2 < _TRIPS // 2 - 1)
        def _():
            dma_start(c0 + 2, hbuf0, segbuf0, semh0, sems0)

        dma_wait(c0 + 1, hbuf1, segbuf1, semh1, sems1)

        @pl.when(base + (c0 + 1) * _CR < end)
        def _():
            process(c0 + 1, hbuf1, segbuf1)

        return 0

    lax.fori_loop(0, _TRIPS // 2, ring, 0)

    pltpu.sync_copy(m_tab, m_hbm.at[wid])
    pltpu.sync_copy(s_tab, s_hbm.at[wid])
    pltpu.sync_copy(acc_tab, acc_hbm.at[wid])


def _combine_body(m_ref, s_ref, acc_ref, out_ref):
    m = m_ref[:, :_NSEG]
    mg = jnp.max(m, axis=0)
    al = jnp.exp(m - mg[None, :])
    s = jnp.sum(s_ref[:, :_NSEG] * al, axis=0)
    acc = jnp.zeros((_NSEG, _HID), jnp.float32)
    for i in range(_NW):
        acc = acc + acc_ref[i] * al[i][:, None]
    out_ref[...] = jnp.where(s[:, None] > 0, acc / s[:, None], 0.0)


def kernel(h, segment_ids, attention_query, w):
    seg = segment_ids.astype(jnp.int32)
    wflat = w.reshape(2 * _HID)
    hflat = h.reshape(_N * _HID)

    mesh = plsc.VectorSubcoreMesh(core_axis_name="c", subcore_axis_name="s",
                                  num_cores=2, num_subcores=16)
    sck = functools.partial(
        pl.kernel,
        mesh=mesh,
        compiler_params=pltpu.CompilerParams(needs_layout_passes=False),
        out_type=[
            jax.ShapeDtypeStruct((_NW, _HID), jnp.float32),
            jax.ShapeDtypeStruct((_NW, _HID), jnp.float32),
            jax.ShapeDtypeStruct((_NW, _NSEG * _HID), jnp.float32),
        ],
        scratch_types=[
            pltpu.VMEM((_CR * _HID,), jnp.float32),
            pltpu.VMEM((_CR * _HID,), jnp.float32),
            pltpu.VMEM((_CR,), jnp.int32),
            pltpu.VMEM((_CR,), jnp.int32),
            pltpu.VMEM((_NSEG * _HID,), jnp.float32),
            pltpu.VMEM((2 * _HID,), jnp.float32),
            pltpu.VMEM((_NSEG,), jnp.float32),
            pltpu.VMEM((_HID,), jnp.float32),
            pltpu.VMEM((_HID,), jnp.float32),
            pltpu.VMEM((_NSEG * _HID,), jnp.float32),
            pltpu.SemaphoreType.DMA,
            pltpu.SemaphoreType.DMA,
            pltpu.SemaphoreType.DMA,
            pltpu.SemaphoreType.DMA,
        ],
    )(_sc_body)
    m_p, s_p, acc_p = sck(hflat, seg, attention_query.reshape(_NSEG * _HID), wflat)

    return pl.pallas_call(
        _combine_body,
        out_shape=jax.ShapeDtypeStruct((_NSEG, _HID), jnp.float32),
    )(m_p, s_p, acc_p.reshape(_NW, _NSEG, _HID))


# speedup vs baseline: 1.7475x; 1.2964x over previous
import functools
import jax
import jax.numpy as jnp
from jax import lax
from jax.experimental import pallas as pl
from jax.experimental.pallas import tpu as pltpu
from jax.experimental.pallas import tpu_sc as plsc

_N = 100000
_HID = 128
_NSEG = 64
_NW = 32
_RW = 3128
_CR = 128
_TRIPS = 26
_NEG = -3.0e38


def _sc_body(h_hbm, seg_hbm, q_hbm, w_hbm, m_hbm, s_hbm, acc_hbm,
             hbuf0, hbuf1, hbuf2, segbuf0, segbuf1, qv, wv, qdotv,
             m_tab, s_tab, acc_tab, semh0, semh1, sems0, sems1):
    wid = lax.axis_index("s") * 2 + lax.axis_index("c")
    base = wid * _RW
    end = jnp.minimum(base + _RW, _N)
    iota = lax.broadcasted_iota(jnp.int32, (16,), 0)
    zero16 = jnp.zeros((16,), jnp.float32)
    neg16 = jnp.full((16,), _NEG, jnp.float32)
    lane0 = iota == 0

    pltpu.sync_copy(w_hbm, wv)
    pltpu.sync_copy(q_hbm, qv)

    def qdot_grp(t, _):
        ridx = (t * 16 + iota) * _HID
        a0 = zero16
        a1 = zero16
        for k16 in range(_HID // 16):
            wvec = wv[pl.ds(_HID + k16 * 16, 16)]
            for dk in range(16):
                k = k16 * 16 + dk
                col = plsc.load_gather(qv, [ridx + k])
                if k % 2 == 0:
                    a0 = a0 + col * jnp.full((16,), wvec[dk])
                else:
                    a1 = a1 + col * jnp.full((16,), wvec[dk])
        qdotv[pl.ds(t * 16, 16)] = a0 + a1
        return 0

    lax.fori_loop(0, _NSEG // 16, qdot_grp, 0)

    for t in range(8):
        m_tab[pl.ds(t * 16, 16)] = neg16
        s_tab[pl.ds(t * 16, 16)] = zero16

    def acc_init(r, _):
        acc_tab[pl.ds(r * 16, 16)] = zero16
        return 0

    lax.fori_loop(0, _NSEG * _HID // 16, acc_init, 0)

    def dma_start(c, hbuf, segbuf, semh, sems):
        lo = base + c * _CR
        st = jnp.minimum(lo, _N - _CR)
        pltpu.async_copy(h_hbm.at[pl.ds(st * _HID, _CR * _HID)], hbuf, semh)
        pltpu.async_copy(seg_hbm.at[pl.ds(st, _CR)], segbuf, sems)

    def dma_wait(c, hbuf, segbuf, semh, sems):
        lo = base + c * _CR
        st = jnp.minimum(lo, _N - _CR)
        pltpu.make_async_copy(h_hbm.at[pl.ds(st * _HID, _CR * _HID)], hbuf, semh).wait()
        pltpu.make_async_copy(seg_hbm.at[pl.ds(st, _CR)], segbuf, sems).wait()

    def process(c, hbuf, segbuf):
        lo = base + c * _CR
        st = jnp.minimum(lo, _N - _CR)

        def spread(r2, _):
            for rr in range(2):
                r = r2 * 2 + rr
                for k in range(_HID // 16):
                    hbuf2[pl.ds(r * (_HID + 1) + k * 16, 16)] = (
                        hbuf[pl.ds(r * _HID + k * 16, 16)])
            return 0

        lax.fori_loop(0, _CR // 2, spread, 0)

        def grp(t, _):
            rows = t * 16 + iota
            ridx = rows * (_HID + 1)
            a0 = zero16
            a1 = zero16
            a2 = zero16
            a3 = zero16
            for k16 in range(_HID // 16):
                wvec = wv[pl.ds(k16 * 16, 16)]
                for dk in range(16):
                    k = k16 * 16 + dk
                    col = plsc.load_gather(hbuf2, [ridx + k])
                    wk = jnp.full((16,), wvec[dk])
                    if k % 4 == 0:
                        a0 = a0 + col * wk
                    elif k % 4 == 1:
                        a1 = a1 + col * wk
                    elif k % 4 == 2:
                        a2 = a2 + col * wk
                    else:
                        a3 = a3 + col * wk
            seg16 = segbuf[pl.ds(t * 16, 16)]
            qd16 = plsc.load_gather(qdotv, [seg16])
            g16 = (a0 + a1) + (a2 + a3) + qd16
            g16 = jnp.where(g16 >= 0, g16, 0.01 * g16)
            gl = st + rows
            valid = (gl >= lo) & (gl < end)
            g16 = jnp.where(valid, g16, _NEG)
            vmask = jnp.where(valid, 1.0, 0.0)

            s0 = seg16[0]
            s15 = seg16[15]

            def fast(_):
                mvec = m_tab[pl.ds(s0, 16)]
                m_old = mvec[0]
                m_new = jnp.maximum(m_old, jnp.max(g16))
                av = jnp.exp(jnp.full((16,), m_old - m_new))
                e16 = jnp.exp(g16 - jnp.full((16,), m_new)) * vmask
                svec = s_tab[pl.ds(s0, 16)]
                s_tab[pl.ds(s0, 16)] = jnp.where(
                    lane0, svec * av + jnp.full((16,), jnp.sum(e16)), svec)
                m_tab[pl.ds(s0, 16)] = jnp.where(
                    lane0, jnp.full((16,), m_new), mvec)
                arow = s0 * _HID
                tmp = [zero16 for _ in range(_HID // 16)]
                for j in range(16):
                    ev = jnp.full((16,), e16[j])
                    hrow = (t * 16 + j) * _HID
                    for k in range(_HID // 16):
                        tmp[k] = tmp[k] + hbuf[pl.ds(hrow + k * 16, 16)] * ev
                for k in range(_HID // 16):
                    acc_tab[pl.ds(arow + k * 16, 16)] = (
                        acc_tab[pl.ds(arow + k * 16, 16)] * av + tmp[k])
                return 0

            def slow(_):
                for j in range(16):
                    sj = seg16[j]
                    gj = g16[j]
                    mvec = m_tab[pl.ds(sj, 16)]
                    m_old = mvec[0]
                    m_new = jnp.maximum(m_old, gj)
                    av = jnp.exp(jnp.full((16,), m_old - m_new))
                    ev = (jnp.exp(jnp.full((16,), gj - m_new))
                          * jnp.full((16,), vmask[j]))
                    svec = s_tab[pl.ds(sj, 16)]
                    s_tab[pl.ds(sj, 16)] = jnp.where(
                        lane0, svec * av + ev, svec)
                    m_tab[pl.ds(sj, 16)] = jnp.where(
                        lane0, jnp.full((16,), m_new), mvec)
                    arow = sj * _HID
                    hrow = (t * 16 + j) * _HID
                    for k in range(_HID // 16):
                        acc_tab[pl.ds(arow + k * 16, 16)] = (
                            acc_tab[pl.ds(arow + k * 16, 16)] * av
                            + hbuf[pl.ds(hrow + k * 16, 16)] * ev)
                return 0

            lax.cond(s0 == s15, fast, slow, 0)
            return 0

        lax.fori_loop(0, _CR // 16, grp, 0)

    dma_start(0, hbuf0, segbuf0, semh0, sems0)

    def ring(c2, _):
        c0 = 2 * c2
        dma_start(c0 + 1, hbuf1, segbuf1, semh1, sems1)
        dma_wait(c0, hbuf0, segbuf0, semh0, sems0)

        @pl.when(base + c0 * _CR < end)
        def _():
            process(c0, hbuf0, segbuf0)

        @pl.when(c2 < _TRIPS // 2 - 1)
        def _():
            dma_start(c0 + 2, hbuf0, segbuf0, semh0, sems0)

        dma_wait(c0 + 1, hbuf1, segbuf1, semh1, sems1)

        @pl.when(base + (c0 + 1) * _CR < end)
        def _():
            process(c0 + 1, hbuf1, segbuf1)

        return 0

    lax.fori_loop(0, _TRIPS // 2, ring, 0)

    pltpu.sync_copy(m_tab, m_hbm.at[wid])
    pltpu.sync_copy(s_tab, s_hbm.at[wid])
    pltpu.sync_copy(acc_tab, acc_hbm.at[wid])


def _combine_body(m_ref, s_ref, acc_ref, out_ref):
    m = m_ref[:, :_NSEG]
    mg = jnp.max(m, axis=0)
    al = jnp.exp(m - mg[None, :])
    s = jnp.sum(s_ref[:, :_NSEG] * al, axis=0)
    acc = jnp.zeros((_NSEG, _HID), jnp.float32)
    for i in range(_NW):
        acc = acc + acc_ref[i] * al[i][:, None]
    out_ref[...] = jnp.where(s[:, None] > 0, acc / s[:, None], 0.0)


def kernel(h, segment_ids, attention_query, w):
    seg = segment_ids.astype(jnp.int32)
    wflat = w.reshape(2 * _HID)
    hflat = h.reshape(_N * _HID)

    mesh = plsc.VectorSubcoreMesh(core_axis_name="c", subcore_axis_name="s",
                                  num_cores=2, num_subcores=16)
    sck = functools.partial(
        pl.kernel,
        mesh=mesh,
        compiler_params=pltpu.CompilerParams(needs_layout_passes=False),
        out_type=[
            jax.ShapeDtypeStruct((_NW, _HID), jnp.float32),
            jax.ShapeDtypeStruct((_NW, _HID), jnp.float32),
            jax.ShapeDtypeStruct((_NW, _NSEG * _HID), jnp.float32),
        ],
        scratch_types=[
            pltpu.VMEM((_CR * _HID,), jnp.float32),
            pltpu.VMEM((_CR * _HID,), jnp.float32),
            pltpu.VMEM((_CR * (_HID + 1),), jnp.float32),
            pltpu.VMEM((_CR,), jnp.int32),
            pltpu.VMEM((_CR,), jnp.int32),
            pltpu.VMEM((_NSEG * _HID,), jnp.float32),
            pltpu.VMEM((2 * _HID,), jnp.float32),
            pltpu.VMEM((_NSEG,), jnp.float32),
            pltpu.VMEM((_HID,), jnp.float32),
            pltpu.VMEM((_HID,), jnp.float32),
            pltpu.VMEM((_NSEG * _HID,), jnp.float32),
            pltpu.SemaphoreType.DMA,
            pltpu.SemaphoreType.DMA,
            pltpu.SemaphoreType.DMA,
            pltpu.SemaphoreType.DMA,
        ],
    )(_sc_body)
    m_p, s_p, acc_p = sck(hflat, seg, attention_query.reshape(_NSEG * _HID), wflat)

    return pl.pallas_call(
        _combine_body,
        out_shape=jax.ShapeDtypeStruct((_NSEG, _HID), jnp.float32),
    )(m_p, s_p, acc_p.reshape(_NW, _NSEG, _HID))
